# baseline (speedup 1.0000x reference)
import math

import jax
import jax.numpy as jnp
from jax import lax
from jax.experimental import pallas as pl
from jax.experimental.pallas import tpu as pltpu
from jax.experimental.pallas import tpu_sc as plsc

VOCAB_SIZE = 1000000
EMB_SIZE = 64
BATCH = 4096
SEQ_LEN = 200

NUM_CORES = 2
NUM_SUBCORES = 16
NUM_WORKERS = NUM_CORES * NUM_SUBCORES
SEQ_PER_WORKER = BATCH // NUM_WORKERS
LANES = 16
SCALE = math.sqrt(EMB_SIZE)


def _body(tok_hbm, emb_hbm, pe_hbm, out_hbm,
          pe_v, idx_v, rows_v, gsem0, gsem1, osem0, osem1):
    cid = lax.axis_index("c")
    sid = lax.axis_index("s")
    wid = sid * NUM_CORES + cid
    base = wid * SEQ_PER_WORKER

    gsem = (gsem0, gsem1)
    osem = (osem0, osem1)

    pltpu.sync_copy(pe_hbm, pe_v)
    pltpu.sync_copy(tok_hbm.at[pl.ds(base, SEQ_PER_WORKER)], idx_v)

    def fire(cc, b):
        pltpu.async_copy(emb_hbm.at[idx_v.at[cc]], rows_v.at[b], gsem[b])

    def step(cc, b, do_owait, do_fire):
        if do_owait:
            pltpu.make_async_copy(rows_v.at[1 - b], out_hbm.at[base],
                                  osem[1 - b]).wait()
        if do_fire:
            fire(cc + 1, 1 - b)
        pltpu.make_async_copy(emb_hbm.at[pl.ds(0, SEQ_LEN)], rows_v.at[b],
                              gsem[b]).wait()

        @pl.loop(0, SEQ_LEN, unroll=4)
        def _pos_loop(p):
            for j in range(EMB_SIZE // LANES):
                sl = pl.ds(j * LANES, LANES)
                rows_v[b, p, sl] = rows_v[b, p, sl] * SCALE + pe_v[p, sl]

        pltpu.async_copy(rows_v.at[b], out_hbm.at[base + cc], osem[b])

    fire(0, 0)
    step(0, 0, do_owait=False, do_fire=True)

    @pl.loop(1, SEQ_PER_WORKER - 1, step=2)
    def _seq_loop(c):
        step(c, 1, do_owait=True, do_fire=True)
        step(c + 1, 0, do_owait=True, do_fire=True)

    step(SEQ_PER_WORKER - 1, 1, do_owait=True, do_fire=False)
    pltpu.make_async_copy(rows_v.at[1], out_hbm.at[base], osem[1]).wait()


def kernel(tokens, embedding, positional_embedding):
    pe = positional_embedding[0, :SEQ_LEN]

    run = pl.kernel(
        _body,
        out_type=jax.ShapeDtypeStruct((BATCH, SEQ_LEN, EMB_SIZE), jnp.float32),
        mesh=plsc.VectorSubcoreMesh(core_axis_name="c", subcore_axis_name="s"),
        compiler_params=pltpu.CompilerParams(use_tc_tiling_on_sc=False),
        scratch_types=[
            pltpu.VMEM((SEQ_LEN, EMB_SIZE), jnp.float32),
            pltpu.VMEM((SEQ_PER_WORKER, SEQ_LEN), jnp.int32),
            pltpu.VMEM((2, SEQ_LEN, EMB_SIZE), jnp.float32),
            pltpu.SemaphoreType.DMA,
            pltpu.SemaphoreType.DMA,
            pltpu.SemaphoreType.DMA,
            pltpu.SemaphoreType.DMA,
        ],
    )
    return run(tokens.astype(jnp.int32), embedding, pe)


# speedup vs baseline: 1.2459x; 1.2459x over previous
import math

import jax
import jax.numpy as jnp
from jax import lax
from jax.experimental import pallas as pl
from jax.experimental.pallas import tpu as pltpu
from jax.experimental.pallas import tpu_sc as plsc

VOCAB_SIZE = 1000000
EMB_SIZE = 64
BATCH = 4096
SEQ_LEN = 200

NUM_CORES = 2
NUM_SUBCORES = 16
NUM_WORKERS = NUM_CORES * NUM_SUBCORES
SEQ_PER_WORKER = BATCH // NUM_WORKERS
LANES = 16
SCALE = math.sqrt(EMB_SIZE)


def _body(tok_hbm, emb_hbm, pe_hbm, out_hbm,
          pe_v, idx_v, rows_v, gsem0, gsem1, osem0, osem1):
    cid = lax.axis_index("c")
    sid = lax.axis_index("s")
    wid = sid * NUM_CORES + cid
    base = wid * SEQ_PER_WORKER

    gsem = (gsem0, gsem1)
    osem = (osem0, osem1)

    pltpu.sync_copy(pe_hbm, pe_v)
    pltpu.sync_copy(tok_hbm.at[pl.ds(base, SEQ_PER_WORKER)], idx_v)

    def fire(cc, b):
        pltpu.async_copy(emb_hbm.at[idx_v.at[cc]], rows_v.at[b], gsem[b])

    def step(cc, b, do_owait, do_fire):
        if do_owait:
            pltpu.make_async_copy(rows_v.at[1 - b], out_hbm.at[base],
                                  osem[1 - b]).wait()
        if do_fire:
            fire(cc + 1, 1 - b)
        pltpu.make_async_copy(emb_hbm.at[pl.ds(0, SEQ_LEN)], rows_v.at[b],
                              gsem[b]).wait()

        @pl.loop(0, SEQ_LEN)
        def _pos_loop(p):
            for j in range(EMB_SIZE // LANES):
                sl = pl.ds(j * LANES, LANES)
                rows_v[b, p, sl] = rows_v[b, p, sl] * SCALE + pe_v[p, sl]

        pltpu.async_copy(rows_v.at[b], out_hbm.at[base + cc], osem[b])

    fire(0, 0)
    step(0, 0, do_owait=False, do_fire=True)

    @pl.loop(1, SEQ_PER_WORKER - 1, step=2)
    def _seq_loop(c):
        step(c, 1, do_owait=True, do_fire=True)
        step(c + 1, 0, do_owait=True, do_fire=True)

    step(SEQ_PER_WORKER - 1, 1, do_owait=True, do_fire=False)
    pltpu.make_async_copy(rows_v.at[1], out_hbm.at[base], osem[1]).wait()


def kernel(tokens, embedding, positional_embedding):
    pe = positional_embedding[0, :SEQ_LEN]

    run = pl.kernel(
        _body,
        out_type=jax.ShapeDtypeStruct((BATCH, SEQ_LEN, EMB_SIZE), jnp.float32),
        mesh=plsc.VectorSubcoreMesh(core_axis_name="c", subcore_axis_name="s"),
        compiler_params=pltpu.CompilerParams(use_tc_tiling_on_sc=False),
        scratch_types=[
            pltpu.VMEM((SEQ_LEN, EMB_SIZE), jnp.float32),
            pltpu.VMEM((SEQ_PER_WORKER, SEQ_LEN), jnp.int32),
            pltpu.VMEM((2, SEQ_LEN, EMB_SIZE), jnp.float32),
            pltpu.SemaphoreType.DMA,
            pltpu.SemaphoreType.DMA,
            pltpu.SemaphoreType.DMA,
            pltpu.SemaphoreType.DMA,
        ],
    )
    return run(tokens.astype(jnp.int32), embedding, pe)
